# Initial kernel scaffold; baseline (speedup 1.0000x reference)
import jax
import jax.numpy as jnp
from jax.experimental import pallas as pl


def kernel(x_tile, x_rrnode, ei_t2t, ei_r2t, ei_t2r, W1_t2t, b1_t2t, W1_r2t, b1_r2t, W1_t2r, b1_t2r, W2_t2t, b2_t2t, W2_r2t, b2_r2t, W2_t2r, b2_t2r, Wm1, bm1, Wm2, bm2):
    raise NotImplementedError("write your pallas kernel here")


# speedup vs baseline: 5.9375x; 5.9375x over previous
import functools

import jax
import jax.numpy as jnp
from jax import lax
from jax.experimental import pallas as pl
from jax.experimental.pallas import tpu as pltpu
from jax.experimental.pallas import tpu_sc as plsc

N = 10000
NP = 10240
D = 128
E = 160000
CW = 128
NW = 32
NCW = 40
EP = NW * NCW * CW
RPT = NP // 16

_MESH = dict(core_axis_name="c", subcore_axis_name="s")


@functools.cache
def _make_degrees():
    return functools.partial(
        pl.kernel,
        out_type=jax.ShapeDtypeStruct((2, 6, NP), jnp.float32),
        mesh=plsc.VectorSubcoreMesh(**_MESH),
        scratch_types=[
            pltpu.VMEM((NCW, CW), jnp.int32),
            pltpu.VMEM((CW,), jnp.float32),
            pltpu.VMEM((NP // 16,), jnp.float32),
        ]
        + [pltpu.VMEM_SHARED((NP,), jnp.float32) for _ in range(6)],
    )(_sc_degrees_body)


def _sc_degrees_body(i0, i1, i2, i3, i4, i5, out, idxbuf, ones, zvec,
                     h0, h1, h2, h3, h4, h5):
    cid = lax.axis_index("c")
    sid = lax.axis_index("s")
    wid = sid * 2 + cid
    hists = (h0, h1, h2, h3, h4, h5)
    srcs = (i0, i1, i2, i3, i4, i5)
    zero16 = jnp.zeros((16,), jnp.float32)
    ones16 = jnp.ones((16,), jnp.float32)
    own = sid * (NP // 16)

    for l in range(CW // 16):
        ones[pl.ds(l * 16, 16)] = ones16

    def zv(i, _):
        zvec[pl.ds(i * 16, 16)] = zero16
        return 0
    lax.fori_loop(0, NP // 16 // 16, zv, 0)

    for h in hists:
        pltpu.sync_copy(zvec, h.at[pl.ds(own, NP // 16)])
    plsc.subcore_barrier()

    for k in range(6):
        src, hist = srcs[k], hists[k]
        pltpu.sync_copy(src.at[wid], idxbuf)

        def row(j, _, hist=hist):
            pltpu.sync_copy(ones, hist.at[idxbuf.at[j]], add=True)
            return 0

        lax.fori_loop(0, NCW, row, 0)

    plsc.subcore_barrier()
    for k in range(6):
        pltpu.sync_copy(hists[k].at[pl.ds(own, NP // 16)],
                        out.at[cid, k, pl.ds(own, NP // 16)])


@functools.cache
def _make_segsum(nrel):
    @functools.partial(
        pl.kernel,
        out_type=jax.ShapeDtypeStruct((nrel, 2, NP, D), jnp.float32),
        mesh=plsc.VectorSubcoreMesh(**_MESH),
        scratch_types=[
            pltpu.VMEM((NCW, CW), jnp.int32),
            pltpu.VMEM((NCW, CW), jnp.int32),
            pltpu.VMEM((CW, D), jnp.float32),
            pltpu.VMEM((CW, D), jnp.float32),
            pltpu.VMEM_SHARED((NP, D), jnp.float32),
            pltpu.SemaphoreType.DMA,
        ],
    )
    def seg(*args):
        out = args[3 * nrel]
        sidx, didx, rows, zbuf, acc, sem = args[3 * nrel + 1:]
        cid = lax.axis_index("c")
        sid = lax.axis_index("s")
        wid = sid * 2 + cid
        zero16 = jnp.zeros((16,), jnp.float32)
        own = sid * RPT

        def zrow(i, _):
            def zl(l, _2):
                zbuf[i, pl.ds(l * 16, 16)] = zero16
                return 0
            lax.fori_loop(0, D // 16, zl, 0)
            return 0
        lax.fori_loop(0, CW, zrow, 0)

        def zero_own_acc():
            for j in range(RPT // CW):
                pltpu.sync_copy(zbuf, acc.at[pl.ds(own + j * CW, CW)])

        zero_own_acc()
        plsc.subcore_barrier()

        for r in range(nrel):
            srcp, dstp, xs = args[3 * r], args[3 * r + 1], args[3 * r + 2]
            pltpu.sync_copy(srcp.at[wid], sidx)
            pltpu.sync_copy(dstp.at[wid], didx)

            def chunk(j, _, xs=xs):
                pltpu.async_copy(xs.at[sidx.at[j]], rows, sem).wait()
                pltpu.sync_copy(rows, acc.at[didx.at[j]], add=True)
                return 0

            lax.fori_loop(0, NCW, chunk, 0)

            plsc.subcore_barrier()
            pltpu.sync_copy(acc.at[pl.ds(own, RPT)],
                            out.at[r, cid, pl.ds(own, RPT)])
            if r + 1 < nrel:
                zero_own_acc()
            plsc.subcore_barrier()

    return seg


def _tc_rsqrt(partials):
    rows = 6 * NP // 128

    def body(p_ref, o_ref):
        s = jnp.sum(p_ref[...], axis=0)
        o_ref[...] = lax.rsqrt(jnp.maximum(s, 1.0))

    return pl.pallas_call(
        body,
        out_shape=jax.ShapeDtypeStruct((rows, 128), jnp.float32),
    )(partials)


_BLK = 1024


def _tc_scale(x_tile, x_rr, ro_t2t, ro_t2r, ro_r2t):
    def body(xt, xr, r1, r2, r3, o1, o2, o3):
        o1[...] = xt[...] * r1[...]
        o2[...] = xt[...] * r2[...]
        o3[...] = xr[...] * r3[...]

    xspec = pl.BlockSpec((_BLK, D), lambda i: (i, 0))
    rspec = pl.BlockSpec((_BLK, 1), lambda i: (i, 0))
    return pl.pallas_call(
        body,
        grid=(NP // _BLK,),
        in_specs=[xspec, xspec, rspec, rspec, rspec],
        out_specs=[xspec, xspec, xspec],
        out_shape=[jax.ShapeDtypeStruct((NP, D), jnp.float32)] * 3,
    )(x_tile, x_rr, ro_t2t, ro_t2r, ro_r2t)


def _dot(a, b):
    return jnp.dot(a, b, preferred_element_type=jnp.float32)


def _tc_layer1(p, ri1, ri2, ri3, ro1, ro2, w1, w2, w3, b1, b2, b3):
    def body(pr, di1, di2, di3, do1, do2, w1r, w2r, w3r, b1r, b2r, b3r, o1, o2):
        a1 = (pr[0, 0] + pr[0, 1]) * di1[...]
        a2 = (pr[1, 0] + pr[1, 1]) * di2[...]
        a3 = (pr[2, 0] + pr[2, 1]) * di3[...]
        ht = jnp.maximum(
            _dot(a1, w1r[...]) + b1r[...] + _dot(a2, w2r[...]) + b2r[...], 0.0)
        hr = jnp.maximum(_dot(a3, w3r[...]) + b3r[...], 0.0)
        o1[...] = ht * do1[...]
        o2[...] = hr * do2[...]

    pspec = pl.BlockSpec((3, 2, _BLK, D), lambda i: (0, 0, i, 0))
    rspec = pl.BlockSpec((_BLK, 1), lambda i: (i, 0))
    wspec = pl.BlockSpec((D, D), lambda i: (0, 0))
    bspec = pl.BlockSpec((1, D), lambda i: (0, 0))
    ospec = pl.BlockSpec((_BLK, D), lambda i: (i, 0))
    return pl.pallas_call(
        body,
        grid=(NP // _BLK,),
        in_specs=[pspec] + [rspec] * 5 + [wspec] * 3 + [bspec] * 3,
        out_specs=[ospec, ospec],
        out_shape=[jax.ShapeDtypeStruct((NP, D), jnp.float32)] * 2,
    )(p, ri1, ri2, ri3, ro1, ro2, w1, w2, w3, b1, b2, b3)


def _tc_layer2(p, ri1, ri2, w1, w2, b1, b2, wm1, bm1, wm2, bm2):
    nsteps = NP // _BLK

    def body(pr, di1, di2, w1r, w2r, b1r, b2r, wm1r, bm1r, wm2r, bm2r, o, acc):
        i = pl.program_id(0)
        a1 = (pr[0, 0] + pr[0, 1]) * di1[...]
        a2 = (pr[1, 0] + pr[1, 1]) * di2[...]
        hb = jnp.maximum(
            _dot(a1, w1r[...]) + b1r[...] + _dot(a2, w2r[...]) + b2r[...], 0.0)
        rowid = lax.broadcasted_iota(jnp.int32, (_BLK, 1), 0) + i * _BLK
        hb = jnp.where(rowid < N, hb, 0.0)
        s = jnp.sum(hb, axis=0, keepdims=True)

        @pl.when(i == 0)
        def _():
            acc[...] = s

        @pl.when(i > 0)
        def _():
            acc[...] = acc[...] + s

        @pl.when(i == nsteps - 1)
        def _():
            hg = acc[...] * (1.0 / N)
            h = jnp.maximum(_dot(hg, wm1r[...]) + bm1r[...], 0.0)
            o[...] = _dot(h, wm2r[...]) + bm2r[...]

    pspec = pl.BlockSpec((2, 2, _BLK, D), lambda i: (0, 0, i, 0))
    rspec = pl.BlockSpec((_BLK, 1), lambda i: (i, 0))
    wspec = pl.BlockSpec((D, D), lambda i: (0, 0))
    bspec = pl.BlockSpec((1, D), lambda i: (0, 0))
    return pl.pallas_call(
        body,
        grid=(nsteps,),
        in_specs=[pspec, rspec, rspec, wspec, wspec, bspec, bspec,
                  wspec, bspec, pl.BlockSpec((D, 1), lambda i: (0, 0)),
                  pl.BlockSpec((1, 1), lambda i: (0, 0))],
        out_specs=pl.BlockSpec((1, 1), lambda i: (0, 0)),
        out_shape=jax.ShapeDtypeStruct((1, 1), jnp.float32),
        scratch_shapes=[pltpu.VMEM((1, D), jnp.float32)],
    )(p, ri1, ri2, w1, w2, b1, b2, wm1, bm1, wm2, bm2)


def _prep_edges(row):
    pad = (N + 224 + (jnp.arange(EP - E) % 16)).astype(jnp.int32)
    return jnp.concatenate([row, pad]).reshape(NW, NCW, CW)


def kernel(x_tile, x_rrnode, ei_t2t, ei_r2t, ei_t2r,
           W1_t2t, b1_t2t, W1_r2t, b1_r2t, W1_t2r, b1_t2r,
           W2_t2t, b2_t2t, W2_r2t, b2_r2t, W2_t2r, b2_t2r,
           Wm1, bm1, Wm2, bm2):
    s_t2t = _prep_edges(ei_t2t[0])
    d_t2t = _prep_edges(ei_t2t[1])
    s_r2t = _prep_edges(ei_r2t[0])
    d_r2t = _prep_edges(ei_r2t[1])
    s_t2r = _prep_edges(ei_t2r[0])
    d_t2r = _prep_edges(ei_t2r[1])
    x_tile_p = jnp.pad(x_tile, ((0, NP - N), (0, 0)))
    x_rr_p = jnp.pad(x_rrnode, ((0, NP - N), (0, 0)))

    degp = _make_degrees()(s_t2t, d_t2t, s_r2t, d_r2t, s_t2r, d_t2r)
    rv = _tc_rsqrt(degp.reshape(2, 6 * NP // 128, 128))
    r6 = rv.reshape(6, NP)
    ro_t2t = r6[0].reshape(NP, 1)
    ri_t2t = r6[1].reshape(NP, 1)
    ro_r2t = r6[2].reshape(NP, 1)
    ri_r2t = r6[3].reshape(NP, 1)
    ro_t2r = r6[4].reshape(NP, 1)
    ri_t2r = r6[5].reshape(NP, 1)

    xs_t2t, xs_t2r, xs_r2t = _tc_scale(x_tile_p, x_rr_p, ro_t2t, ro_t2r, ro_r2t)

    p1 = _make_segsum(3)(s_t2t, d_t2t, xs_t2t,
                         s_r2t, d_r2t, xs_r2t,
                         s_t2r, d_t2r, xs_t2r)

    hs_t2t, hs_r2t = _tc_layer1(
        p1, ri_t2t, ri_r2t, ri_t2r, ro_t2t, ro_r2t,
        W1_t2t, W1_r2t, W1_t2r,
        b1_t2t.reshape(1, D), b1_r2t.reshape(1, D), b1_t2r.reshape(1, D))

    p2 = _make_segsum(2)(s_t2t, d_t2t, hs_t2t,
                         s_r2t, d_r2t, hs_r2t)

    return _tc_layer2(
        p2, ri_t2t, ri_r2t, W2_t2t, W2_r2t,
        b2_t2t.reshape(1, D), b2_r2t.reshape(1, D),
        Wm1, bm1.reshape(1, D), Wm2, bm2.reshape(1, 1))


# speedup vs baseline: 7.2179x; 1.2156x over previous
import functools

import jax
import jax.numpy as jnp
from jax import lax
from jax.experimental import pallas as pl
from jax.experimental.pallas import tpu as pltpu
from jax.experimental.pallas import tpu_sc as plsc

N = 10000
NP = 10240
D = 128
E = 160000
CW = 128
NW = 32
NCW = 40
EP = NW * NCW * CW
RPT = NP // 16

_MESH = dict(core_axis_name="c", subcore_axis_name="s")


@functools.cache
def _make_degrees():
    return functools.partial(
        pl.kernel,
        out_type=jax.ShapeDtypeStruct((2, 6, NP), jnp.float32),
        mesh=plsc.VectorSubcoreMesh(**_MESH),
        scratch_types=[
            pltpu.VMEM((NCW, CW), jnp.int32),
            pltpu.VMEM((CW,), jnp.float32),
            pltpu.VMEM((NP // 16,), jnp.float32),
            pltpu.SemaphoreType.DMA,
        ]
        + [pltpu.VMEM_SHARED((NP,), jnp.float32) for _ in range(6)],
    )(_sc_degrees_body)


def _sc_degrees_body(i0, i1, i2, i3, i4, i5, out, idxbuf, ones, zvec, sem,
                     h0, h1, h2, h3, h4, h5):
    cid = lax.axis_index("c")
    sid = lax.axis_index("s")
    wid = sid * 2 + cid
    hists = (h0, h1, h2, h3, h4, h5)
    srcs = (i0, i1, i2, i3, i4, i5)
    zero16 = jnp.zeros((16,), jnp.float32)
    ones16 = jnp.ones((16,), jnp.float32)
    own = sid * (NP // 16)

    for l in range(CW // 16):
        ones[pl.ds(l * 16, 16)] = ones16

    def zv(i, _):
        zvec[pl.ds(i * 16, 16)] = zero16
        return 0
    lax.fori_loop(0, NP // 16 // 16, zv, 0)

    for h in hists:
        pltpu.sync_copy(zvec, h.at[pl.ds(own, NP // 16)])
    plsc.subcore_barrier()

    for k in range(6):
        src, hist = srcs[k], hists[k]
        pltpu.sync_copy(src.at[wid], idxbuf)

        def row(j, _, hist=hist):
            pltpu.sync_copy(ones, hist.at[idxbuf.at[j]], add=True)
            return 0

        lax.fori_loop(0, NCW, row, 0)

    plsc.subcore_barrier()
    for k in range(6):
        pltpu.sync_copy(hists[k].at[pl.ds(own, NP // 16)],
                        out.at[cid, k, pl.ds(own, NP // 16)])


@functools.cache
def _make_segsum(nrel):
    @functools.partial(
        pl.kernel,
        out_type=jax.ShapeDtypeStruct((nrel, 2, NP, D), jnp.float32),
        mesh=plsc.VectorSubcoreMesh(**_MESH),
        scratch_types=[
            pltpu.VMEM((NCW, CW), jnp.int32),
            pltpu.VMEM((NCW, CW), jnp.int32),
            pltpu.VMEM((CW, D), jnp.float32),
            pltpu.VMEM((CW, D), jnp.float32),
            pltpu.VMEM_SHARED((NP, D), jnp.float32),
            pltpu.SemaphoreType.DMA,
            pltpu.SemaphoreType.DMA,
        ],
    )
    def seg(*args):
        out = args[3 * nrel]
        sidx, didx, rows0, rows1, acc, gsem, ssem = args[3 * nrel + 1:]
        rowbufs = (rows0, rows1)
        cid = lax.axis_index("c")
        sid = lax.axis_index("s")
        wid = sid * 2 + cid
        zero16 = jnp.zeros((16,), jnp.float32)
        own = sid * RPT

        def zero_rows0():
            def zrow(i, _):
                def zl(l, _2):
                    rows0[i, pl.ds(l * 16, 16)] = zero16
                    return 0
                lax.fori_loop(0, D // 16, zl, 0)
                return 0
            lax.fori_loop(0, CW, zrow, 0)

        def zero_own_acc():
            for j in range(RPT // CW):
                pltpu.sync_copy(rows0, acc.at[pl.ds(own + j * CW, CW)])

        zero_rows0()
        zero_own_acc()
        plsc.subcore_barrier()

        for r in range(nrel):
            srcp, dstp, xs = args[3 * r], args[3 * r + 1], args[3 * r + 2]
            pltpu.sync_copy(srcp.at[wid], sidx)
            pltpu.sync_copy(dstp.at[wid], didx)

            g = pltpu.async_copy(xs.at[sidx.at[0]], rows0, gsem)
            prev_s = None
            for j in range(NCW):
                g.wait()
                if prev_s is not None:
                    prev_s.wait()
                if j + 1 < NCW:
                    g = pltpu.async_copy(
                        xs.at[sidx.at[j + 1]], rowbufs[(j + 1) % 2], gsem)
                prev_s = pltpu.async_copy(
                    rowbufs[j % 2], acc.at[didx.at[j]], ssem, add=True)
            prev_s.wait()

            plsc.subcore_barrier()
            pltpu.sync_copy(acc.at[pl.ds(own, RPT)],
                            out.at[r, cid, pl.ds(own, RPT)])
            if r + 1 < nrel:
                zero_rows0()
                zero_own_acc()
            plsc.subcore_barrier()

    return seg


def _tc_rsqrt(partials):
    rows = 6 * NP // 128

    def body(p_ref, o_ref):
        s = jnp.sum(p_ref[...], axis=0)
        o_ref[...] = 1.0 / jnp.sqrt(jnp.maximum(s, 1.0))

    return pl.pallas_call(
        body,
        out_shape=jax.ShapeDtypeStruct((rows, 128), jnp.float32),
    )(partials)


_BLK = 1024


def _tc_scale(x_tile, x_rr, ro_t2t, ro_t2r, ro_r2t):
    def body(xt, xr, r1, r2, r3, o1, o2, o3):
        o1[...] = xt[...] * r1[...]
        o2[...] = xt[...] * r2[...]
        o3[...] = xr[...] * r3[...]

    xspec = pl.BlockSpec((_BLK, D), lambda i: (i, 0))
    rspec = pl.BlockSpec((_BLK, 1), lambda i: (i, 0))
    return pl.pallas_call(
        body,
        grid=(NP // _BLK,),
        in_specs=[xspec, xspec, rspec, rspec, rspec],
        out_specs=[xspec, xspec, xspec],
        out_shape=[jax.ShapeDtypeStruct((NP, D), jnp.float32)] * 3,
    )(x_tile, x_rr, ro_t2t, ro_t2r, ro_r2t)


def _dot(a, b):
    return jnp.dot(a.astype(jnp.bfloat16), b.astype(jnp.bfloat16),
                   preferred_element_type=jnp.float32)


def _dot_head(a, b):
    return _dot(a, b)


def _tc_layer1(p, ri1, ri2, ri3, ro1, ro2, w1, w2, w3, b1, b2, b3):
    def body(pr, di1, di2, di3, do1, do2, w1r, w2r, w3r, b1r, b2r, b3r, o1, o2):
        a1 = (pr[0, 0] + pr[0, 1]) * di1[...]
        a2 = (pr[1, 0] + pr[1, 1]) * di2[...]
        a3 = (pr[2, 0] + pr[2, 1]) * di3[...]
        ht = jnp.maximum(
            _dot(a1, w1r[...]) + b1r[...] + _dot(a2, w2r[...]) + b2r[...], 0.0)
        hr = jnp.maximum(_dot(a3, w3r[...]) + b3r[...], 0.0)
        o1[...] = ht * do1[...]
        o2[...] = hr * do2[...]

    pspec = pl.BlockSpec((3, 2, _BLK, D), lambda i: (0, 0, i, 0))
    rspec = pl.BlockSpec((_BLK, 1), lambda i: (i, 0))
    wspec = pl.BlockSpec((D, D), lambda i: (0, 0))
    bspec = pl.BlockSpec((1, D), lambda i: (0, 0))
    ospec = pl.BlockSpec((_BLK, D), lambda i: (i, 0))
    return pl.pallas_call(
        body,
        grid=(NP // _BLK,),
        in_specs=[pspec] + [rspec] * 5 + [wspec] * 3 + [bspec] * 3,
        out_specs=[ospec, ospec],
        out_shape=[jax.ShapeDtypeStruct((NP, D), jnp.float32)] * 2,
    )(p, ri1, ri2, ri3, ro1, ro2, w1, w2, w3, b1, b2, b3)


def _tc_layer2(p, ri1, ri2, w1, w2, b1, b2, wm1, bm1, wm2, bm2):
    nsteps = NP // _BLK

    def body(pr, di1, di2, w1r, w2r, b1r, b2r, wm1r, bm1r, wm2r, bm2r, o, acc):
        i = pl.program_id(0)
        a1 = (pr[0, 0] + pr[0, 1]) * di1[...]
        a2 = (pr[1, 0] + pr[1, 1]) * di2[...]
        hb = jnp.maximum(
            _dot(a1, w1r[...]) + b1r[...] + _dot(a2, w2r[...]) + b2r[...], 0.0)
        rowid = lax.broadcasted_iota(jnp.int32, (_BLK, 1), 0) + i * _BLK
        hb = jnp.where(rowid < N, hb, 0.0)
        s = jnp.sum(hb, axis=0, keepdims=True)

        @pl.when(i == 0)
        def _():
            acc[...] = s

        @pl.when(i > 0)
        def _():
            acc[...] = acc[...] + s

        @pl.when(i == nsteps - 1)
        def _():
            hg = acc[...] / jnp.float32(N)
            h = jnp.maximum(_dot_head(hg, wm1r[...]) + bm1r[...], 0.0)
            o[...] = (jnp.sum(h * wm2r[...], axis=1, keepdims=True)
                      + bm2r[...])

    pspec = pl.BlockSpec((2, 2, _BLK, D), lambda i: (0, 0, i, 0))
    rspec = pl.BlockSpec((_BLK, 1), lambda i: (i, 0))
    wspec = pl.BlockSpec((D, D), lambda i: (0, 0))
    bspec = pl.BlockSpec((1, D), lambda i: (0, 0))
    return pl.pallas_call(
        body,
        grid=(nsteps,),
        in_specs=[pspec, rspec, rspec, wspec, wspec, bspec, bspec,
                  wspec, bspec, bspec,
                  pl.BlockSpec((1, 1), lambda i: (0, 0))],
        out_specs=pl.BlockSpec((1, 1), lambda i: (0, 0)),
        out_shape=jax.ShapeDtypeStruct((1, 1), jnp.float32),
        scratch_shapes=[pltpu.VMEM((1, D), jnp.float32)],
    )(p, ri1, ri2, w1, w2, b1, b2, wm1, bm1, wm2, bm2)


def _prep_edges(row):
    pad = (N + 224 + (jnp.arange(EP - E) % 16)).astype(jnp.int32)
    return jnp.concatenate([row, pad]).reshape(NW, NCW, CW)


def kernel(x_tile, x_rrnode, ei_t2t, ei_r2t, ei_t2r,
           W1_t2t, b1_t2t, W1_r2t, b1_r2t, W1_t2r, b1_t2r,
           W2_t2t, b2_t2t, W2_r2t, b2_r2t, W2_t2r, b2_t2r,
           Wm1, bm1, Wm2, bm2):
    s_t2t = _prep_edges(ei_t2t[0])
    d_t2t = _prep_edges(ei_t2t[1])
    s_r2t = _prep_edges(ei_r2t[0])
    d_r2t = _prep_edges(ei_r2t[1])
    s_t2r = _prep_edges(ei_t2r[0])
    d_t2r = _prep_edges(ei_t2r[1])
    x_tile_p = jnp.pad(x_tile, ((0, NP - N), (0, 0)))
    x_rr_p = jnp.pad(x_rrnode, ((0, NP - N), (0, 0)))

    degp = _make_degrees()(s_t2t, d_t2t, s_r2t, d_r2t, s_t2r, d_t2r)
    rv = _tc_rsqrt(degp.reshape(2, 6 * NP // 128, 128))
    r6 = rv.reshape(6, NP)
    ro_t2t = r6[0].reshape(NP, 1)
    ri_t2t = r6[1].reshape(NP, 1)
    ro_r2t = r6[2].reshape(NP, 1)
    ri_r2t = r6[3].reshape(NP, 1)
    ro_t2r = r6[4].reshape(NP, 1)
    ri_t2r = r6[5].reshape(NP, 1)

    xs_t2t, xs_t2r, xs_r2t = _tc_scale(x_tile_p, x_rr_p, ro_t2t, ro_t2r, ro_r2t)

    p1 = _make_segsum(3)(s_t2t, d_t2t, xs_t2t,
                         s_r2t, d_r2t, xs_r2t,
                         s_t2r, d_t2r, xs_t2r)

    hs_t2t, hs_r2t = _tc_layer1(
        p1, ri_t2t, ri_r2t, ri_t2r, ro_t2t, ro_r2t,
        W1_t2t, W1_r2t, W1_t2r,
        b1_t2t.reshape(1, D), b1_r2t.reshape(1, D), b1_t2r.reshape(1, D))

    p2 = _make_segsum(2)(s_t2t, d_t2t, hs_t2t,
                         s_r2t, d_r2t, hs_r2t)

    return _tc_layer2(
        p2, ri_t2t, ri_r2t, W2_t2t, W2_r2t,
        b2_t2t.reshape(1, D), b2_r2t.reshape(1, D),
        Wm1, bm1.reshape(1, D), Wm2.reshape(1, D), bm2.reshape(1, 1))


# speedup vs baseline: 7.4270x; 1.0290x over previous
import functools

import jax
import jax.numpy as jnp
from jax import lax
from jax.experimental import pallas as pl
from jax.experimental.pallas import tpu as pltpu
from jax.experimental.pallas import tpu_sc as plsc

N = 10000
NP = 10240
D = 128
E = 160000
CW = 128
NW = 32
NCW = 40
EP = NW * NCW * CW
RPT = NP // 16

_MESH = dict(core_axis_name="c", subcore_axis_name="s")


@functools.cache
def _make_degrees():
    return functools.partial(
        pl.kernel,
        out_type=jax.ShapeDtypeStruct((2, 6, NP), jnp.float32),
        mesh=plsc.VectorSubcoreMesh(**_MESH),
        scratch_types=[
            pltpu.VMEM((NCW, CW), jnp.int32),
            pltpu.VMEM((CW,), jnp.float32),
            pltpu.VMEM((NP // 16,), jnp.float32),
            pltpu.SemaphoreType.DMA,
        ]
        + [pltpu.VMEM_SHARED((NP,), jnp.float32) for _ in range(6)],
    )(_sc_degrees_body)


def _sc_degrees_body(idx, out, idxbuf, ones, zvec, sem,
                     h0, h1, h2, h3, h4, h5):
    cid = lax.axis_index("c")
    sid = lax.axis_index("s")
    wid = sid * 2 + cid
    hists = (h0, h1, h2, h3, h4, h5)
    zero16 = jnp.zeros((16,), jnp.float32)
    ones16 = jnp.ones((16,), jnp.float32)
    own = sid * (NP // 16)

    for l in range(CW // 16):
        ones[pl.ds(l * 16, 16)] = ones16

    def zv(i, _):
        zvec[pl.ds(i * 16, 16)] = zero16
        return 0
    lax.fori_loop(0, NP // 16 // 16, zv, 0)

    for h in hists:
        pltpu.sync_copy(zvec, h.at[pl.ds(own, NP // 16)])
    plsc.subcore_barrier()

    for k in range(6):
        hist = hists[k]
        pltpu.sync_copy(idx.at[k, wid], idxbuf)

        def row(j, _, hist=hist):
            pltpu.sync_copy(ones, hist.at[idxbuf.at[j]], add=True)
            return 0

        lax.fori_loop(0, NCW, row, 0)

    plsc.subcore_barrier()
    for k in range(6):
        pltpu.sync_copy(hists[k].at[pl.ds(own, NP // 16)],
                        out.at[cid, k, pl.ds(own, NP // 16)])


@functools.cache
def _make_segsum(nrel):
    ZR = 40

    @functools.partial(
        pl.kernel,
        out_type=jax.ShapeDtypeStruct((nrel, 2, NP, D), jnp.float32),
        mesh=plsc.VectorSubcoreMesh(**_MESH),
        scratch_types=[
            pltpu.VMEM((NCW, CW), jnp.int32),
            pltpu.VMEM((NCW, CW), jnp.int32),
            pltpu.VMEM((CW, D), jnp.float32),
            pltpu.VMEM((CW, D), jnp.float32),
            pltpu.VMEM((ZR, D), jnp.float32),
            pltpu.VMEM_SHARED((NP, D), jnp.float32),
            pltpu.SemaphoreType.DMA,
            pltpu.SemaphoreType.DMA,
        ],
    )
    def seg(*args):
        idx = args[0]
        out = args[1 + nrel]
        sidx, didx, rows0, rows1, zbuf, acc, gsem, ssem = args[2 + nrel:]
        rowbufs = (rows0, rows1)
        cid = lax.axis_index("c")
        sid = lax.axis_index("s")
        wid = sid * 2 + cid
        zero16 = jnp.zeros((16,), jnp.float32)
        own = sid * RPT

        def zrow(i, _):
            def zl(l, _2):
                zbuf[i, pl.ds(l * 16, 16)] = zero16
                return 0
            lax.fori_loop(0, D // 16, zl, 0)
            return 0
        lax.fori_loop(0, ZR, zrow, 0)

        def zero_own_acc():
            for j in range(RPT // ZR):
                pltpu.sync_copy(zbuf, acc.at[pl.ds(own + j * ZR, ZR)])

        for r in range(nrel):
            xs = args[1 + r]
            pltpu.sync_copy(idx.at[2 * r, wid], sidx)
            pltpu.sync_copy(idx.at[2 * r + 1, wid], didx)
            g0 = pltpu.async_copy(xs.at[sidx.at[0]], rows0, gsem)
            g1 = pltpu.async_copy(xs.at[sidx.at[1]], rows1, gsem)
            gathers = [g0, g1]
            if r == 0:
                zero_own_acc()
            else:
                pltpu.sync_copy(acc.at[pl.ds(own, RPT)],
                                out.at[r - 1, cid, pl.ds(own, RPT)])
                zero_own_acc()
            plsc.subcore_barrier()

            prev_s = None
            for j in range(NCW):
                gathers[j].wait()
                if prev_s is not None:
                    prev_s.wait()
                if j >= 1 and j + 1 < NCW:
                    gathers.append(pltpu.async_copy(
                        xs.at[sidx.at[j + 1]], rowbufs[(j + 1) % 2], gsem))
                prev_s = pltpu.async_copy(
                    rowbufs[j % 2], acc.at[didx.at[j]], ssem, add=True)
            prev_s.wait()
            plsc.subcore_barrier()

        pltpu.sync_copy(acc.at[pl.ds(own, RPT)],
                        out.at[nrel - 1, cid, pl.ds(own, RPT)])

    return seg


def _tc_rsqrt(partials):
    rows = 6 * NP // 128

    def body(p_ref, o_ref):
        s = jnp.sum(p_ref[...], axis=0)
        o_ref[...] = 1.0 / jnp.sqrt(jnp.maximum(s, 1.0))

    return pl.pallas_call(
        body,
        out_shape=jax.ShapeDtypeStruct((rows, 128), jnp.float32),
    )(partials)


_BLK = 1024


def _tc_scale(x_tile, x_rr, ro_t2t, ro_t2r, ro_r2t):
    def body(xt, xr, r1, r2, r3, o1, o2, o3):
        o1[...] = xt[...] * r1[...]
        o2[...] = xt[...] * r2[...]
        o3[...] = xr[...] * r3[...]

    xspec = pl.BlockSpec((_BLK, D), lambda i: (i, 0))
    rspec = pl.BlockSpec((_BLK, 1), lambda i: (i, 0))
    return pl.pallas_call(
        body,
        grid=(NP // _BLK,),
        in_specs=[xspec, xspec, rspec, rspec, rspec],
        out_specs=[xspec, xspec, xspec],
        out_shape=[jax.ShapeDtypeStruct((NP, D), jnp.float32)] * 3,
    )(x_tile, x_rr, ro_t2t, ro_t2r, ro_r2t)


def _dot(a, b):
    return jnp.dot(a.astype(jnp.bfloat16), b.astype(jnp.bfloat16),
                   preferred_element_type=jnp.float32)


def _dot_head(a, b):
    return _dot(a, b)


def _tc_layer1(p, ri1, ri2, ri3, ro1, ro2, w1, w2, w3, b1, b2, b3):
    def body(pr, di1, di2, di3, do1, do2, w1r, w2r, w3r, b1r, b2r, b3r, o1, o2):
        a1 = (pr[0, 0] + pr[0, 1]) * di1[...]
        a2 = (pr[1, 0] + pr[1, 1]) * di2[...]
        a3 = (pr[2, 0] + pr[2, 1]) * di3[...]
        ht = jnp.maximum(
            _dot(a1, w1r[...]) + b1r[...] + _dot(a2, w2r[...]) + b2r[...], 0.0)
        hr = jnp.maximum(_dot(a3, w3r[...]) + b3r[...], 0.0)
        o1[...] = ht * do1[...]
        o2[...] = hr * do2[...]

    pspec = pl.BlockSpec((3, 2, _BLK, D), lambda i: (0, 0, i, 0))
    rspec = pl.BlockSpec((_BLK, 1), lambda i: (i, 0))
    wspec = pl.BlockSpec((D, D), lambda i: (0, 0))
    bspec = pl.BlockSpec((1, D), lambda i: (0, 0))
    ospec = pl.BlockSpec((_BLK, D), lambda i: (i, 0))
    return pl.pallas_call(
        body,
        grid=(NP // _BLK,),
        in_specs=[pspec] + [rspec] * 5 + [wspec] * 3 + [bspec] * 3,
        out_specs=[ospec, ospec],
        out_shape=[jax.ShapeDtypeStruct((NP, D), jnp.float32)] * 2,
    )(p, ri1, ri2, ri3, ro1, ro2, w1, w2, w3, b1, b2, b3)


def _tc_layer2(p, ri1, ri2, w1, w2, b1, b2, wm1, bm1, wm2, bm2):
    nsteps = NP // _BLK

    def body(pr, di1, di2, w1r, w2r, b1r, b2r, wm1r, bm1r, wm2r, bm2r, o, acc):
        i = pl.program_id(0)
        a1 = (pr[0, 0] + pr[0, 1]) * di1[...]
        a2 = (pr[1, 0] + pr[1, 1]) * di2[...]
        hb = jnp.maximum(
            _dot(a1, w1r[...]) + b1r[...] + _dot(a2, w2r[...]) + b2r[...], 0.0)
        rowid = lax.broadcasted_iota(jnp.int32, (_BLK, 1), 0) + i * _BLK
        hb = jnp.where(rowid < N, hb, 0.0)
        s = jnp.sum(hb, axis=0, keepdims=True)

        @pl.when(i == 0)
        def _():
            acc[...] = s

        @pl.when(i > 0)
        def _():
            acc[...] = acc[...] + s

        @pl.when(i == nsteps - 1)
        def _():
            hg = acc[...] / jnp.float32(N)
            h = jnp.maximum(_dot_head(hg, wm1r[...]) + bm1r[...], 0.0)
            o[...] = (jnp.sum(h * wm2r[...], axis=1, keepdims=True)
                      + bm2r[...])

    pspec = pl.BlockSpec((2, 2, _BLK, D), lambda i: (0, 0, i, 0))
    rspec = pl.BlockSpec((_BLK, 1), lambda i: (i, 0))
    wspec = pl.BlockSpec((D, D), lambda i: (0, 0))
    bspec = pl.BlockSpec((1, D), lambda i: (0, 0))
    return pl.pallas_call(
        body,
        grid=(nsteps,),
        in_specs=[pspec, rspec, rspec, wspec, wspec, bspec, bspec,
                  wspec, bspec, bspec,
                  pl.BlockSpec((1, 1), lambda i: (0, 0))],
        out_specs=pl.BlockSpec((1, 1), lambda i: (0, 0)),
        out_shape=jax.ShapeDtypeStruct((1, 1), jnp.float32),
        scratch_shapes=[pltpu.VMEM((1, D), jnp.float32)],
    )(p, ri1, ri2, w1, w2, b1, b2, wm1, bm1, wm2, bm2)


def _prep_edges(ei_t2t, ei_r2t, ei_t2r):
    stk = jnp.concatenate([ei_t2t, ei_r2t, ei_t2r], axis=0)
    pad = (N + 224 + (jnp.arange(EP - E) % 16)).astype(jnp.int32)
    pad6 = jnp.broadcast_to(pad, (6, EP - E))
    return jnp.concatenate([stk, pad6], axis=1).reshape(6, NW, NCW, CW)


def kernel(x_tile, x_rrnode, ei_t2t, ei_r2t, ei_t2r,
           W1_t2t, b1_t2t, W1_r2t, b1_r2t, W1_t2r, b1_t2r,
           W2_t2t, b2_t2t, W2_r2t, b2_r2t, W2_t2r, b2_t2r,
           Wm1, bm1, Wm2, bm2):
    idx_all = _prep_edges(ei_t2t, ei_r2t, ei_t2r)
    x_tile_p = jnp.pad(x_tile, ((0, NP - N), (0, 0)))
    x_rr_p = jnp.pad(x_rrnode, ((0, NP - N), (0, 0)))

    degp = _make_degrees()(idx_all)
    rv = _tc_rsqrt(degp.reshape(2, 6 * NP // 128, 128))
    r6 = rv.reshape(6, NP)
    ro_t2t = r6[0].reshape(NP, 1)
    ri_t2t = r6[1].reshape(NP, 1)
    ro_r2t = r6[2].reshape(NP, 1)
    ri_r2t = r6[3].reshape(NP, 1)
    ro_t2r = r6[4].reshape(NP, 1)
    ri_t2r = r6[5].reshape(NP, 1)

    xs_t2t, xs_t2r, xs_r2t = _tc_scale(x_tile_p, x_rr_p, ro_t2t, ro_t2r, ro_r2t)

    p1 = _make_segsum(3)(idx_all, xs_t2t, xs_r2t, xs_t2r)

    hs_t2t, hs_r2t = _tc_layer1(
        p1, ri_t2t, ri_r2t, ri_t2r, ro_t2t, ro_r2t,
        W1_t2t, W1_r2t, W1_t2r,
        b1_t2t.reshape(1, D), b1_r2t.reshape(1, D), b1_t2r.reshape(1, D))

    p2 = _make_segsum(2)(idx_all, hs_t2t, hs_r2t)

    return _tc_layer2(
        p2, ri_t2t, ri_r2t, W2_t2t, W2_r2t,
        b2_t2t.reshape(1, D), b2_r2t.reshape(1, D),
        Wm1, bm1.reshape(1, D), Wm2.reshape(1, D), bm2.reshape(1, 1))


# speedup vs baseline: 7.5676x; 1.0189x over previous
import functools

import jax
import jax.numpy as jnp
from jax import lax
from jax.experimental import pallas as pl
from jax.experimental.pallas import tpu as pltpu
from jax.experimental.pallas import tpu_sc as plsc

N = 10000
NP = 10240
D = 128
E = 160000
CW = 128
NW = 32
NCW = 40
EP = NW * NCW * CW
RPT = NP // 16

_MESH = dict(core_axis_name="c", subcore_axis_name="s")


@functools.cache
def _make_degrees():
    return functools.partial(
        pl.kernel,
        out_type=jax.ShapeDtypeStruct((2, 6, NP), jnp.float32),
        mesh=plsc.VectorSubcoreMesh(**_MESH),
        scratch_types=[
            pltpu.VMEM((NCW, CW), jnp.int32),
            pltpu.VMEM((CW,), jnp.float32),
            pltpu.VMEM((NP // 16,), jnp.float32),
            pltpu.SemaphoreType.DMA,
        ]
        + [pltpu.VMEM_SHARED((NP,), jnp.float32) for _ in range(6)],
    )(_sc_degrees_body)


def _sc_degrees_body(idx, out, idxbuf, ones, zvec, sem,
                     h0, h1, h2, h3, h4, h5):
    cid = lax.axis_index("c")
    sid = lax.axis_index("s")
    wid = sid * 2 + cid
    hists = (h0, h1, h2, h3, h4, h5)
    zero16 = jnp.zeros((16,), jnp.float32)
    ones16 = jnp.ones((16,), jnp.float32)
    own = sid * (NP // 16)

    for l in range(CW // 16):
        ones[pl.ds(l * 16, 16)] = ones16

    def zv(i, _):
        zvec[pl.ds(i * 16, 16)] = zero16
        return 0
    lax.fori_loop(0, NP // 16 // 16, zv, 0)

    zs = [pltpu.async_copy(zvec, h.at[pl.ds(own, NP // 16)], sem)
          for h in hists]
    for z in zs:
        z.wait()
    plsc.subcore_barrier()

    for k in range(6):
        hist = hists[k]
        pltpu.sync_copy(idx.at[k, wid], idxbuf)

        def row(j, _, hist=hist):
            pltpu.sync_copy(ones, hist.at[idxbuf.at[j]], add=True)
            return 0

        lax.fori_loop(0, NCW, row, 0)

    plsc.subcore_barrier()
    ws = [pltpu.async_copy(hists[k].at[pl.ds(own, NP // 16)],
                           out.at[cid, k, pl.ds(own, NP // 16)], sem)
          for k in range(6)]
    for w in ws:
        w.wait()


@functools.cache
def _make_segsum(nrel):
    ZR = 40

    @functools.partial(
        pl.kernel,
        out_type=jax.ShapeDtypeStruct((nrel, 2, NP, D), jnp.float32),
        mesh=plsc.VectorSubcoreMesh(**_MESH),
        scratch_types=[
            pltpu.VMEM((NCW, CW), jnp.int32),
            pltpu.VMEM((NCW, CW), jnp.int32),
            pltpu.VMEM((CW, D), jnp.float32),
            pltpu.VMEM((CW, D), jnp.float32),
            pltpu.VMEM((ZR, D), jnp.float32),
            pltpu.VMEM_SHARED((NP, D), jnp.float32),
            pltpu.SemaphoreType.DMA,
            pltpu.SemaphoreType.DMA,
        ],
    )
    def seg(*args):
        idx = args[0]
        out = args[1 + nrel]
        sidx, didx, rows0, rows1, zbuf, acc, gsem, ssem = args[2 + nrel:]
        rowbufs = (rows0, rows1)
        cid = lax.axis_index("c")
        sid = lax.axis_index("s")
        wid = sid * 2 + cid
        zero16 = jnp.zeros((16,), jnp.float32)
        own = sid * RPT

        def zrow(i, _):
            def zl(l, _2):
                zbuf[i, pl.ds(l * 16, 16)] = zero16
                return 0
            lax.fori_loop(0, D // 16, zl, 0)
            return 0
        lax.fori_loop(0, ZR, zrow, 0)

        def zero_own_acc():
            zs = [pltpu.async_copy(zbuf, acc.at[pl.ds(own + j * ZR, ZR)], ssem)
                  for j in range(RPT // ZR)]
            for z in zs:
                z.wait()

        for r in range(nrel):
            xs = args[1 + r]
            i0 = pltpu.async_copy(idx.at[2 * r, wid], sidx, gsem)
            i1 = pltpu.async_copy(idx.at[2 * r + 1, wid], didx, gsem)
            i0.wait()
            i1.wait()
            g0 = pltpu.async_copy(xs.at[sidx.at[0]], rows0, gsem)
            g1 = pltpu.async_copy(xs.at[sidx.at[1]], rows1, gsem)
            gathers = [g0, g1]
            if r == 0:
                zero_own_acc()
            else:
                pltpu.sync_copy(acc.at[pl.ds(own, RPT)],
                                out.at[r - 1, cid, pl.ds(own, RPT)])
                zero_own_acc()
            plsc.subcore_barrier()

            prev_s = None
            for j in range(NCW):
                gathers[j].wait()
                if prev_s is not None:
                    prev_s.wait()
                if j >= 1 and j + 1 < NCW:
                    gathers.append(pltpu.async_copy(
                        xs.at[sidx.at[j + 1]], rowbufs[(j + 1) % 2], gsem))
                prev_s = pltpu.async_copy(
                    rowbufs[j % 2], acc.at[didx.at[j]], ssem, add=True)
            prev_s.wait()
            plsc.subcore_barrier()

        pltpu.sync_copy(acc.at[pl.ds(own, RPT)],
                        out.at[nrel - 1, cid, pl.ds(own, RPT)])

    return seg


def _tc_rsqrt(partials):
    rows = 6 * NP // 128

    def body(p_ref, o_ref):
        s = jnp.sum(p_ref[...], axis=0)
        o_ref[...] = 1.0 / jnp.sqrt(jnp.maximum(s, 1.0))

    return pl.pallas_call(
        body,
        out_shape=jax.ShapeDtypeStruct((rows, 128), jnp.float32),
    )(partials)


_BLK = 1024


def _tc_scale(x_tile, x_rr, ro_t2t, ro_t2r, ro_r2t):
    def body(xt, xr, r1, r2, r3, o1, o2, o3):
        o1[...] = xt[...] * r1[...]
        o2[...] = xt[...] * r2[...]
        o3[...] = xr[...] * r3[...]

    xspec = pl.BlockSpec((_BLK, D), lambda i: (i, 0))
    rspec = pl.BlockSpec((_BLK, 1), lambda i: (i, 0))
    return pl.pallas_call(
        body,
        grid=(NP // _BLK,),
        in_specs=[xspec, xspec, rspec, rspec, rspec],
        out_specs=[xspec, xspec, xspec],
        out_shape=[jax.ShapeDtypeStruct((NP, D), jnp.float32)] * 3,
    )(x_tile, x_rr, ro_t2t, ro_t2r, ro_r2t)


def _dot(a, b):
    return jnp.dot(a.astype(jnp.bfloat16), b.astype(jnp.bfloat16),
                   preferred_element_type=jnp.float32)


def _dot_head(a, b):
    return _dot(a, b)


def _tc_layer1(p, ri1, ri2, ri3, ro1, ro2, w1, w2, w3, b1, b2, b3):
    def body(pr, di1, di2, di3, do1, do2, w1r, w2r, w3r, b1r, b2r, b3r, o1, o2):
        a1 = (pr[0, 0] + pr[0, 1]) * di1[...]
        a2 = (pr[1, 0] + pr[1, 1]) * di2[...]
        a3 = (pr[2, 0] + pr[2, 1]) * di3[...]
        ht = jnp.maximum(
            _dot(a1, w1r[...]) + b1r[...] + _dot(a2, w2r[...]) + b2r[...], 0.0)
        hr = jnp.maximum(_dot(a3, w3r[...]) + b3r[...], 0.0)
        o1[...] = ht * do1[...]
        o2[...] = hr * do2[...]

    pspec = pl.BlockSpec((3, 2, _BLK, D), lambda i: (0, 0, i, 0))
    rspec = pl.BlockSpec((_BLK, 1), lambda i: (i, 0))
    wspec = pl.BlockSpec((D, D), lambda i: (0, 0))
    bspec = pl.BlockSpec((1, D), lambda i: (0, 0))
    ospec = pl.BlockSpec((_BLK, D), lambda i: (i, 0))
    return pl.pallas_call(
        body,
        grid=(NP // _BLK,),
        in_specs=[pspec] + [rspec] * 5 + [wspec] * 3 + [bspec] * 3,
        out_specs=[ospec, ospec],
        out_shape=[jax.ShapeDtypeStruct((NP, D), jnp.float32)] * 2,
    )(p, ri1, ri2, ri3, ro1, ro2, w1, w2, w3, b1, b2, b3)


def _tc_layer2(p, ri1, ri2, w1, w2, b1, b2, wm1, bm1, wm2, bm2):
    nsteps = NP // _BLK

    def body(pr, di1, di2, w1r, w2r, b1r, b2r, wm1r, bm1r, wm2r, bm2r, o, acc):
        i = pl.program_id(0)
        a1 = (pr[0, 0] + pr[0, 1]) * di1[...]
        a2 = (pr[1, 0] + pr[1, 1]) * di2[...]
        hb = jnp.maximum(
            _dot(a1, w1r[...]) + b1r[...] + _dot(a2, w2r[...]) + b2r[...], 0.0)
        rowid = lax.broadcasted_iota(jnp.int32, (_BLK, 1), 0) + i * _BLK
        hb = jnp.where(rowid < N, hb, 0.0)
        s = jnp.sum(hb, axis=0, keepdims=True)

        @pl.when(i == 0)
        def _():
            acc[...] = s

        @pl.when(i > 0)
        def _():
            acc[...] = acc[...] + s

        @pl.when(i == nsteps - 1)
        def _():
            hg = acc[...] / jnp.float32(N)
            h = jnp.maximum(_dot_head(hg, wm1r[...]) + bm1r[...], 0.0)
            o[...] = (jnp.sum(h * wm2r[...], axis=1, keepdims=True)
                      + bm2r[...])

    pspec = pl.BlockSpec((2, 2, _BLK, D), lambda i: (0, 0, i, 0))
    rspec = pl.BlockSpec((_BLK, 1), lambda i: (i, 0))
    wspec = pl.BlockSpec((D, D), lambda i: (0, 0))
    bspec = pl.BlockSpec((1, D), lambda i: (0, 0))
    return pl.pallas_call(
        body,
        grid=(nsteps,),
        in_specs=[pspec, rspec, rspec, wspec, wspec, bspec, bspec,
                  wspec, bspec, bspec,
                  pl.BlockSpec((1, 1), lambda i: (0, 0))],
        out_specs=pl.BlockSpec((1, 1), lambda i: (0, 0)),
        out_shape=jax.ShapeDtypeStruct((1, 1), jnp.float32),
        scratch_shapes=[pltpu.VMEM((1, D), jnp.float32)],
    )(p, ri1, ri2, w1, w2, b1, b2, wm1, bm1, wm2, bm2)


def _prep_edges(ei_t2t, ei_r2t, ei_t2r):
    stk = jnp.concatenate([ei_t2t, ei_r2t, ei_t2r], axis=0)
    pad = (N + 224 + (jnp.arange(EP - E) % 16)).astype(jnp.int32)
    pad6 = jnp.broadcast_to(pad, (6, EP - E))
    return jnp.concatenate([stk, pad6], axis=1).reshape(6, NW, NCW, CW)


def kernel(x_tile, x_rrnode, ei_t2t, ei_r2t, ei_t2r,
           W1_t2t, b1_t2t, W1_r2t, b1_r2t, W1_t2r, b1_t2r,
           W2_t2t, b2_t2t, W2_r2t, b2_r2t, W2_t2r, b2_t2r,
           Wm1, bm1, Wm2, bm2):
    idx_all = _prep_edges(ei_t2t, ei_r2t, ei_t2r)
    x_tile_p = jnp.pad(x_tile, ((0, NP - N), (0, 0)))
    x_rr_p = jnp.pad(x_rrnode, ((0, NP - N), (0, 0)))

    degp = _make_degrees()(idx_all)
    rv = _tc_rsqrt(degp.reshape(2, 6 * NP // 128, 128))
    r6 = rv.reshape(6, NP)
    ro_t2t = r6[0].reshape(NP, 1)
    ri_t2t = r6[1].reshape(NP, 1)
    ro_r2t = r6[2].reshape(NP, 1)
    ri_r2t = r6[3].reshape(NP, 1)
    ro_t2r = r6[4].reshape(NP, 1)
    ri_t2r = r6[5].reshape(NP, 1)

    xs_t2t, xs_t2r, xs_r2t = _tc_scale(x_tile_p, x_rr_p, ro_t2t, ro_t2r, ro_r2t)

    p1 = _make_segsum(3)(idx_all, xs_t2t, xs_r2t, xs_t2r)

    hs_t2t, hs_r2t = _tc_layer1(
        p1, ri_t2t, ri_r2t, ri_t2r, ro_t2t, ro_r2t,
        W1_t2t, W1_r2t, W1_t2r,
        b1_t2t.reshape(1, D), b1_r2t.reshape(1, D), b1_t2r.reshape(1, D))

    p2 = _make_segsum(2)(idx_all, hs_t2t, hs_r2t)

    return _tc_layer2(
        p2, ri_t2t, ri_r2t, W2_t2t, W2_r2t,
        b2_t2t.reshape(1, D), b2_r2t.reshape(1, D),
        Wm1, bm1.reshape(1, D), Wm2.reshape(1, D), bm2.reshape(1, 1))
